# baseline (speedup 1.0000x reference)
import functools

import jax
import jax.numpy as jnp
from jax import lax
from jax.experimental import pallas as pl
from jax.experimental.pallas import tpu as pltpu
from jax.experimental.pallas import tpu_sc as plsc

B = 4096
D_IN = 210
N_LATENTS = 32768
K_SPARSITY = 100

BM0 = 256
BME, LNE = 256, 2048
BM3, LN3 = 256, 2048


def _norm_body(x_ref, pb_ref, xc_ref, mu_ref, std_ref):
    x = x_ref[...]
    mu = jnp.mean(x, axis=1, keepdims=True)
    std = jnp.sqrt(jnp.mean((x - mu) ** 2, axis=1, keepdims=True))
    xn = (x - mu) / (std + 1e-5)
    xc_ref[...] = xn - pb_ref[...]
    mu_ref[...] = mu
    std_ref[...] = std


def _enc_body(xc_ref, w_ref, b_ref, out_ref):
    out_ref[...] = (
        jnp.dot(xc_ref[...], w_ref[...], preferred_element_type=jnp.float32)
        + b_ref[...]
    )


def _sc_select_body(pre_hbm, th_hbm, row_v, cm_v, cnt_v, base_v, cid_v,
                    cand_v, hist_v, th_v, *, rows_per_w, num_cores):
    wid = lax.axis_index("s") * num_cores + lax.axis_index("c")
    base = wid * rows_per_w
    iota = lax.iota(jnp.int32, 16)
    lane_base = iota * 256
    ones = jnp.ones((16,), jnp.int32)
    zeros16 = jnp.zeros((16,), jnp.int32)
    nvr = N_LATENTS // 16
    ncv = nvr // 16

    def scan_bins(rank):
        above = jnp.int32(0)
        T = jnp.int32(0)
        above_sel = jnp.int32(0)
        for g in range(15, -1, -1):
            acc = zeros16
            for l in range(16):
                acc = acc + hist_v[pl.ds(l * 256 + g * 16, 16)]
            pc = plsc.cumsum(acc)
            gtot = jnp.max(pc)
            dc = gtot - pc + acc
            cond = (above + dc) >= rank
            lstar = jnp.sum(cond.astype(jnp.int32)) - 1
            in_group = jnp.logical_and(above < rank, above + gtot >= rank)
            dcl = jnp.sum(jnp.where(iota == lstar, dc, 0))
            tvl = jnp.sum(jnp.where(iota == lstar, acc, 0))
            T = jnp.where(in_group, g * 16 + lstar, T)
            above_sel = jnp.where(in_group, above + dcl - tvl, above_sel)
            above = above + gtot
        return T, above_sel

    def radix_select(nv, load_keys, k_rank, levels):
        rank = k_rank
        want = jnp.int32(0)
        for li, shift in enumerate((24, 16, 8, 0)[:levels]):
            @plsc.parallel_loop(0, 256, unroll=8)
            def _zh(i):
                hist_v[pl.ds(i * 16, 16)] = zeros16

            @plsc.parallel_loop(0, nv, unroll=4)
            def _p(j, li=li, shift=shift, want=want):
                key = load_keys(j)
                if li == 0:
                    b = lax.shift_right_arithmetic(key, 24) + 128
                    plsc.addupdate_scatter(hist_v, [lane_base + b], ones)
                else:
                    m = lax.shift_right_arithmetic(key, shift + 8) == want
                    b = lax.shift_right_arithmetic(key, shift) & 255
                    plsc.addupdate_scatter(hist_v, [lane_base + b], ones,
                                           mask=m)

            tn, above_n = scan_bins(rank)
            rank = rank - above_n
            want = tn - 128 if li == 0 else lax.shift_left(want, 8) | tn
        return want

    def process_row(r, c):
        pltpu.sync_copy(pre_hbm.at[base + r], row_v)

        @plsc.parallel_loop(0, nvr, unroll=8)
        def _ck(j):
            v = row_v[pl.ds(j * 16, 16)]
            t = plsc.bitcast(v, jnp.int32)
            key = t ^ (lax.shift_right_arithmetic(t, 31)
                       & jnp.int32(0x7FFFFFFF))
            row_v[pl.ds(j * 16, 16)] = plsc.bitcast(key, jnp.float32)
            mx = jnp.full((16,), jnp.max(key), jnp.int32)
            plsc.store_scatter(cm_v, [jnp.full((16,), j, jnp.int32)], mx,
                               mask=(iota == 0))

        def load_cm(j):
            return cm_v[pl.ds(j * 16, 16)]
        want_a = radix_select(ncv, load_cm, K_SPARSITY, 2)
        mhat = lax.shift_left(want_a, 16)

        @plsc.parallel_loop(0, ncv, unroll=8)
        def _pc(j):
            m = cm_v[pl.ds(j * 16, 16)] >= mhat
            cnt = jnp.full((16,), jnp.sum(m.astype(jnp.int32)), jnp.int32)
            plsc.store_scatter(cnt_v, [jnp.full((16,), j, jnp.int32)], cnt,
                               mask=(iota == 0))

        carry = jnp.int32(0)
        for v in range(ncv // 16):
            cv = cnt_v[pl.ds(v * 16, 16)]
            pc = plsc.cumsum(cv)
            base_v[pl.ds(v * 16, 16)] = pc - cv + carry
            carry = carry + jnp.max(pc)
        n_cand = carry

        @plsc.parallel_loop(0, ncv, unroll=4)
        def _cc(j):
            ck = cm_v[pl.ds(j * 16, 16)]
            m = ck >= mhat
            cs = plsc.cumsum(m.astype(jnp.int32))
            bj = plsc.load_gather(base_v, [jnp.full((16,), j, jnp.int32)])
            plsc.store_scatter(cid_v, [bj + cs - 1], j * 16 + iota, mask=m)

        @plsc.parallel_loop(0, n_cand, unroll=2)
        def _g(i):
            cid = plsc.load_gather(cid_v, [jnp.full((16,), i, jnp.int32)])
            keys = plsc.load_gather(row_v, [cid * 16 + iota])
            cand_v[pl.ds(i * 16, 16)] = keys

        def load_cand(j):
            return plsc.bitcast(cand_v[pl.ds(j * 16, 16)], jnp.int32)
        want = radix_select(n_cand, load_cand, K_SPARSITY, 4)

        kv = jnp.full((16,), want, dtype=jnp.int32)
        fv = plsc.bitcast(
            kv ^ (lax.shift_right_arithmetic(kv, 31) & jnp.int32(0x7FFFFFFF)),
            jnp.float32)
        plsc.store_scatter(th_v, [jnp.full((16,), r, jnp.int32)], fv,
                           mask=(iota == 0))
        return c

    lax.fori_loop(0, rows_per_w, process_row, 0)
    pltpu.sync_copy(th_v, th_hbm.at[pl.ds(base, rows_per_w)])


def _finish_body(pre_ref, th_ref, w_ref, pb_ref, mu_ref, std_ref,
                 lat_ref, rec_ref, *, n_lat_blocks):
    l = pl.program_id(1)
    pre = pre_ref[...]
    lat = jnp.where(pre >= th_ref[...], pre, 0.0)
    lat_ref[...] = lat
    part = jnp.dot(lat, w_ref[...], preferred_element_type=jnp.float32)

    @pl.when(l == 0)
    def _():
        rec_ref[...] = part

    @pl.when(l > 0)
    def _():
        rec_ref[...] += part

    @pl.when(l == n_lat_blocks - 1)
    def _():
        rec_ref[...] = (rec_ref[...] + pb_ref[...]) * std_ref[...] + mu_ref[...]


def kernel(pos, vel, acc, root_lin_vel, root_ang_vel, root_lin_acc,
           root_ang_acc, joint_centers, root_pos_history, root_euler_history,
           pre_bias, latent_bias, W_enc, W_dec):
    x = jnp.concatenate([
        pos, vel, acc, root_lin_vel, root_ang_vel, root_lin_acc, root_ang_acc,
        joint_centers, root_pos_history, root_euler_history,
    ], axis=-1)
    b = x.shape[0]
    pb = pre_bias.reshape(1, D_IN)
    lb = latent_bias.reshape(1, N_LATENTS)

    xc, mu, std = pl.pallas_call(
        _norm_body,
        grid=(b // BM0,),
        in_specs=[
            pl.BlockSpec((BM0, D_IN), lambda i: (i, 0)),
            pl.BlockSpec((1, D_IN), lambda i: (0, 0)),
        ],
        out_specs=[
            pl.BlockSpec((BM0, D_IN), lambda i: (i, 0)),
            pl.BlockSpec((BM0, 1), lambda i: (i, 0)),
            pl.BlockSpec((BM0, 1), lambda i: (i, 0)),
        ],
        out_shape=[
            jax.ShapeDtypeStruct((b, D_IN), jnp.float32),
            jax.ShapeDtypeStruct((b, 1), jnp.float32),
            jax.ShapeDtypeStruct((b, 1), jnp.float32),
        ],
    )(x, pb)

    pre_act = pl.pallas_call(
        _enc_body,
        grid=(N_LATENTS // LNE, b // BME),
        in_specs=[
            pl.BlockSpec((BME, D_IN), lambda l, i: (i, 0)),
            pl.BlockSpec((D_IN, LNE), lambda l, i: (0, l)),
            pl.BlockSpec((1, LNE), lambda l, i: (0, l)),
        ],
        out_specs=pl.BlockSpec((BME, LNE), lambda l, i: (i, l)),
        out_shape=jax.ShapeDtypeStruct((b, N_LATENTS), jnp.float32),
    )(xc, W_enc, lb)

    info = plsc.get_sparse_core_info()
    nw = info.num_cores * info.num_subcores
    rows_per_w = b // nw
    sel = pl.kernel(
        functools.partial(_sc_select_body, rows_per_w=rows_per_w,
                          num_cores=info.num_cores),
        out_type=jax.ShapeDtypeStruct((b,), jnp.float32),
        mesh=plsc.VectorSubcoreMesh(core_axis_name="c",
                                    subcore_axis_name="s"),
        compiler_params=pltpu.CompilerParams(needs_layout_passes=False),
        scratch_types=[
            pltpu.VMEM((N_LATENTS,), jnp.float32),
            pltpu.VMEM((N_LATENTS // 16,), jnp.int32),
            pltpu.VMEM((N_LATENTS // 256,), jnp.int32),
            pltpu.VMEM((N_LATENTS // 256,), jnp.int32),
            pltpu.VMEM((N_LATENTS // 16,), jnp.int32),
            pltpu.VMEM((N_LATENTS,), jnp.float32),
            pltpu.VMEM((4096,), jnp.int32),
            pltpu.VMEM((rows_per_w,), jnp.float32),
        ],
    )
    thresh = sel(pre_act).reshape(b, 1)

    n_lat_blocks = N_LATENTS // LN3
    latents, recons = pl.pallas_call(
        functools.partial(_finish_body, n_lat_blocks=n_lat_blocks),
        grid=(b // BM3, n_lat_blocks),
        in_specs=[
            pl.BlockSpec((BM3, LN3), lambda i, l: (i, l)),
            pl.BlockSpec((BM3, 1), lambda i, l: (i, 0)),
            pl.BlockSpec((LN3, D_IN), lambda i, l: (l, 0)),
            pl.BlockSpec((1, D_IN), lambda i, l: (0, 0)),
            pl.BlockSpec((BM3, 1), lambda i, l: (i, 0)),
            pl.BlockSpec((BM3, 1), lambda i, l: (i, 0)),
        ],
        out_specs=[
            pl.BlockSpec((BM3, LN3), lambda i, l: (i, l)),
            pl.BlockSpec((BM3, D_IN), lambda i, l: (i, 0)),
        ],
        out_shape=[
            jax.ShapeDtypeStruct((b, N_LATENTS), jnp.float32),
            jax.ShapeDtypeStruct((b, D_IN), jnp.float32),
        ],
    )(pre_act, thresh, W_dec, pb, mu, std)

    return pre_act, latents, recons


# speedup vs baseline: 1.0306x; 1.0306x over previous
import functools

import jax
import jax.numpy as jnp
from jax import lax
from jax.experimental import pallas as pl
from jax.experimental.pallas import tpu as pltpu
from jax.experimental.pallas import tpu_sc as plsc

B = 4096
D_IN = 210
N_LATENTS = 32768
K_SPARSITY = 100

BM0 = 256
BME, LNE = 256, 2048
BM3, LN3 = 256, 2048


def _norm_body(x_ref, pb_ref, xc_ref, mu_ref, std_ref):
    x = x_ref[...]
    mu = jnp.mean(x, axis=1, keepdims=True)
    std = jnp.sqrt(jnp.mean((x - mu) ** 2, axis=1, keepdims=True))
    xn = (x - mu) / (std + 1e-5)
    xc_ref[...] = xn - pb_ref[...]
    mu_ref[...] = mu
    std_ref[...] = std


def _enc_body(xc_ref, w_ref, b_ref, out_ref):
    out_ref[...] = (
        jnp.dot(xc_ref[...], w_ref[...], preferred_element_type=jnp.float32)
        + b_ref[...]
    )


def _sc_select_body(pre_hbm, th_hbm, row_v, cm_v, cnt_v, base_v, cid_v,
                    cand_v, hist_v, th_v, *, rows_per_w, num_cores):
    wid = lax.axis_index("s") * num_cores + lax.axis_index("c")
    base = wid * rows_per_w
    iota = lax.iota(jnp.int32, 16)
    lane_base = iota * 256
    ones = jnp.ones((16,), jnp.int32)
    zeros16 = jnp.zeros((16,), jnp.int32)
    nvr = N_LATENTS // 16
    ncv = nvr // 16

    def scan_bins(rank):
        above = jnp.int32(0)
        T = jnp.int32(0)
        above_sel = jnp.int32(0)
        for g in range(15, -1, -1):
            acc = zeros16
            for l in range(16):
                acc = acc + hist_v[pl.ds(l * 256 + g * 16, 16)]
            pc = plsc.cumsum(acc)
            gtot = jnp.max(pc)
            dc = gtot - pc + acc
            cond = (above + dc) >= rank
            lstar = jnp.sum(cond.astype(jnp.int32)) - 1
            in_group = jnp.logical_and(above < rank, above + gtot >= rank)
            dcl = jnp.sum(jnp.where(iota == lstar, dc, 0))
            tvl = jnp.sum(jnp.where(iota == lstar, acc, 0))
            T = jnp.where(in_group, g * 16 + lstar, T)
            above_sel = jnp.where(in_group, above + dcl - tvl, above_sel)
            above = above + gtot
        return T, above_sel

    def radix_select(nv, load_keys, k_rank, levels):
        rank = k_rank
        want = jnp.int32(0)
        for li, shift in enumerate((24, 16, 8, 0)[:levels]):
            @plsc.parallel_loop(0, 256, unroll=8)
            def _zh(i):
                hist_v[pl.ds(i * 16, 16)] = zeros16

            @plsc.parallel_loop(0, nv, unroll=4)
            def _p(j, li=li, shift=shift, want=want):
                key = load_keys(j)
                if li == 0:
                    b = lax.shift_right_arithmetic(key, 24) + 128
                    plsc.addupdate_scatter(hist_v, [lane_base + b], ones)
                else:
                    m = lax.shift_right_arithmetic(key, shift + 8) == want
                    b = lax.shift_right_arithmetic(key, shift) & 255
                    plsc.addupdate_scatter(hist_v, [lane_base + b], ones,
                                           mask=m)

            tn, above_n = scan_bins(rank)
            rank = rank - above_n
            want = tn - 128 if li == 0 else lax.shift_left(want, 8) | tn
        return want

    def process_row(r, c):
        pltpu.sync_copy(pre_hbm.at[base + r], row_v)

        @plsc.parallel_loop(0, ncv, unroll=2)
        def _ck(j):
            mx = jnp.full((16,), jnp.int32(-2147483648))
            for l in range(16):
                v = row_v[pl.ds(j * 16 + l * nvr, 16)]
                t = plsc.bitcast(v, jnp.int32)
                key = t ^ (lax.shift_right_arithmetic(t, 31)
                           & jnp.int32(0x7FFFFFFF))
                row_v[pl.ds(j * 16 + l * nvr, 16)] = plsc.bitcast(
                    key, jnp.float32)
                mx = jnp.maximum(mx, key)
            cm_v[pl.ds(j * 16, 16)] = mx

        def load_cm(j):
            return cm_v[pl.ds(j * 16, 16)]
        want_a = radix_select(ncv, load_cm, K_SPARSITY, 2)
        mhat = lax.shift_left(want_a, 16)

        @plsc.parallel_loop(0, ncv, unroll=8)
        def _pc(j):
            m = cm_v[pl.ds(j * 16, 16)] >= mhat
            cnt = jnp.full((16,), jnp.sum(m.astype(jnp.int32)), jnp.int32)
            plsc.store_scatter(cnt_v, [jnp.full((16,), j, jnp.int32)], cnt,
                               mask=(iota == 0))

        carry = jnp.int32(0)
        for v in range(ncv // 16):
            cv = cnt_v[pl.ds(v * 16, 16)]
            pc = plsc.cumsum(cv)
            base_v[pl.ds(v * 16, 16)] = pc - cv + carry
            carry = carry + jnp.max(pc)
        n_cand = carry

        @plsc.parallel_loop(0, ncv, unroll=4)
        def _cc(j):
            ck = cm_v[pl.ds(j * 16, 16)]
            m = ck >= mhat
            cs = plsc.cumsum(m.astype(jnp.int32))
            bj = plsc.load_gather(base_v, [jnp.full((16,), j, jnp.int32)])
            plsc.store_scatter(cid_v, [bj + cs - 1], j * 16 + iota, mask=m)

        @plsc.parallel_loop(0, n_cand, unroll=2)
        def _g(i):
            cid = plsc.load_gather(cid_v, [jnp.full((16,), i, jnp.int32)])
            keys = plsc.load_gather(row_v, [cid + iota * nvr])
            cand_v[pl.ds(i * 16, 16)] = keys

        def load_cand(j):
            return plsc.bitcast(cand_v[pl.ds(j * 16, 16)], jnp.int32)
        want = radix_select(n_cand, load_cand, K_SPARSITY, 4)

        kv = jnp.full((16,), want, dtype=jnp.int32)
        fv = plsc.bitcast(
            kv ^ (lax.shift_right_arithmetic(kv, 31) & jnp.int32(0x7FFFFFFF)),
            jnp.float32)
        plsc.store_scatter(th_v, [jnp.full((16,), r, jnp.int32)], fv,
                           mask=(iota == 0))
        return c

    lax.fori_loop(0, rows_per_w, process_row, 0)
    pltpu.sync_copy(th_v, th_hbm.at[pl.ds(base, rows_per_w)])


def _finish_body(pre_ref, th_ref, w_ref, pb_ref, mu_ref, std_ref,
                 lat_ref, rec_ref, *, n_lat_blocks):
    l = pl.program_id(1)
    pre = pre_ref[...]
    lat = jnp.where(pre >= th_ref[...], pre, 0.0)
    lat_ref[...] = lat
    part = jnp.dot(lat, w_ref[...], preferred_element_type=jnp.float32)

    @pl.when(l == 0)
    def _():
        rec_ref[...] = part

    @pl.when(l > 0)
    def _():
        rec_ref[...] += part

    @pl.when(l == n_lat_blocks - 1)
    def _():
        rec_ref[...] = (rec_ref[...] + pb_ref[...]) * std_ref[...] + mu_ref[...]


def kernel(pos, vel, acc, root_lin_vel, root_ang_vel, root_lin_acc,
           root_ang_acc, joint_centers, root_pos_history, root_euler_history,
           pre_bias, latent_bias, W_enc, W_dec):
    x = jnp.concatenate([
        pos, vel, acc, root_lin_vel, root_ang_vel, root_lin_acc, root_ang_acc,
        joint_centers, root_pos_history, root_euler_history,
    ], axis=-1)
    b = x.shape[0]
    pb = pre_bias.reshape(1, D_IN)
    lb = latent_bias.reshape(1, N_LATENTS)

    xc, mu, std = pl.pallas_call(
        _norm_body,
        grid=(b // BM0,),
        in_specs=[
            pl.BlockSpec((BM0, D_IN), lambda i: (i, 0)),
            pl.BlockSpec((1, D_IN), lambda i: (0, 0)),
        ],
        out_specs=[
            pl.BlockSpec((BM0, D_IN), lambda i: (i, 0)),
            pl.BlockSpec((BM0, 1), lambda i: (i, 0)),
            pl.BlockSpec((BM0, 1), lambda i: (i, 0)),
        ],
        out_shape=[
            jax.ShapeDtypeStruct((b, D_IN), jnp.float32),
            jax.ShapeDtypeStruct((b, 1), jnp.float32),
            jax.ShapeDtypeStruct((b, 1), jnp.float32),
        ],
    )(x, pb)

    pre_act = pl.pallas_call(
        _enc_body,
        grid=(N_LATENTS // LNE, b // BME),
        in_specs=[
            pl.BlockSpec((BME, D_IN), lambda l, i: (i, 0)),
            pl.BlockSpec((D_IN, LNE), lambda l, i: (0, l)),
            pl.BlockSpec((1, LNE), lambda l, i: (0, l)),
        ],
        out_specs=pl.BlockSpec((BME, LNE), lambda l, i: (i, l)),
        out_shape=jax.ShapeDtypeStruct((b, N_LATENTS), jnp.float32),
    )(xc, W_enc, lb)

    info = plsc.get_sparse_core_info()
    nw = info.num_cores * info.num_subcores
    rows_per_w = b // nw
    sel = pl.kernel(
        functools.partial(_sc_select_body, rows_per_w=rows_per_w,
                          num_cores=info.num_cores),
        out_type=jax.ShapeDtypeStruct((b,), jnp.float32),
        mesh=plsc.VectorSubcoreMesh(core_axis_name="c",
                                    subcore_axis_name="s"),
        compiler_params=pltpu.CompilerParams(needs_layout_passes=False),
        scratch_types=[
            pltpu.VMEM((N_LATENTS,), jnp.float32),
            pltpu.VMEM((N_LATENTS // 16,), jnp.int32),
            pltpu.VMEM((N_LATENTS // 256,), jnp.int32),
            pltpu.VMEM((N_LATENTS // 256,), jnp.int32),
            pltpu.VMEM((N_LATENTS // 16,), jnp.int32),
            pltpu.VMEM((N_LATENTS,), jnp.float32),
            pltpu.VMEM((4096,), jnp.int32),
            pltpu.VMEM((rows_per_w,), jnp.float32),
        ],
    )
    thresh = sel(pre_act).reshape(b, 1)

    n_lat_blocks = N_LATENTS // LN3
    latents, recons = pl.pallas_call(
        functools.partial(_finish_body, n_lat_blocks=n_lat_blocks),
        grid=(b // BM3, n_lat_blocks),
        in_specs=[
            pl.BlockSpec((BM3, LN3), lambda i, l: (i, l)),
            pl.BlockSpec((BM3, 1), lambda i, l: (i, 0)),
            pl.BlockSpec((LN3, D_IN), lambda i, l: (l, 0)),
            pl.BlockSpec((1, D_IN), lambda i, l: (0, 0)),
            pl.BlockSpec((BM3, 1), lambda i, l: (i, 0)),
            pl.BlockSpec((BM3, 1), lambda i, l: (i, 0)),
        ],
        out_specs=[
            pl.BlockSpec((BM3, LN3), lambda i, l: (i, l)),
            pl.BlockSpec((BM3, D_IN), lambda i, l: (i, 0)),
        ],
        out_shape=[
            jax.ShapeDtypeStruct((b, N_LATENTS), jnp.float32),
            jax.ShapeDtypeStruct((b, D_IN), jnp.float32),
        ],
    )(pre_act, thresh, W_dec, pb, mu, std)

    return pre_act, latents, recons


# speedup vs baseline: 1.2671x; 1.2295x over previous
import functools

import jax
import jax.numpy as jnp
from jax import lax
from jax.experimental import pallas as pl
from jax.experimental.pallas import tpu as pltpu
from jax.experimental.pallas import tpu_sc as plsc

B = 4096
D_IN = 210
N_LATENTS = 32768
K_SPARSITY = 100

BM0 = 256
BME, LNE = 256, 2048
BM3, LN3 = 256, 2048


def _norm_body(x_ref, pb_ref, xc_ref, mu_ref, std_ref):
    x = x_ref[...]
    mu = jnp.mean(x, axis=1, keepdims=True)
    std = jnp.sqrt(jnp.mean((x - mu) ** 2, axis=1, keepdims=True))
    xn = (x - mu) / (std + 1e-5)
    xc_ref[...] = xn - pb_ref[...]
    mu_ref[...] = mu
    std_ref[...] = std


def _enc_body(xc_ref, w_ref, b_ref, out_ref):
    out_ref[...] = (
        jnp.dot(xc_ref[...], w_ref[...], preferred_element_type=jnp.float32)
        + b_ref[...]
    )


def _sc_select_body(pre_hbm, th_hbm, row_v, cm_v, cnt_v, base_v, cid_v,
                    cand_v, hist_v, tot_v, th_v, *, rows_per_w, num_cores):
    wid = lax.axis_index("s") * num_cores + lax.axis_index("c")
    base = wid * rows_per_w
    iota = lax.iota(jnp.int32, 16)
    lane_base = iota * 256
    ones = jnp.ones((16,), jnp.int32)
    zeros16 = jnp.zeros((16,), jnp.int32)
    nvr = N_LATENTS // 16
    ncv = nvr // 16

    def scan_bins(rank):
        for g in range(16):
            acc = zeros16
            for l in range(16):
                acc = acc + hist_v[pl.ds(l * 256 + g * 16, 16)]
            tot_v[pl.ds(g * 16, 16)] = acc
        trs = [plsc.load_gather(tot_v, [iota * 16 + t]) for t in range(16)]
        vs = [zeros16] * 16
        run = zeros16
        for t in range(15, -1, -1):
            run = run + trs[t]
            vs[t] = run
        pc = plsc.cumsum(vs[0])
        above_vec = jnp.max(pc) - pc
        cnt_vec = zeros16
        sel_vec = zeros16
        for t in range(16):
            cond = (above_vec + vs[t]) >= rank
            cnt_vec = cnt_vec + cond.astype(jnp.int32)
            sel_vec = sel_vec + jnp.where(cond, 0, trs[t])
        T = jnp.sum(cnt_vec) - 1
        above_sel = jnp.sum(sel_vec)
        return T, above_sel

    def radix_select(nv, load_keys, k_rank, levels):
        rank = k_rank
        want = jnp.int32(0)
        for li, shift in enumerate((24, 16, 8, 0)[:levels]):
            @plsc.parallel_loop(0, 256, unroll=8)
            def _zh(i):
                hist_v[pl.ds(i * 16, 16)] = zeros16

            @plsc.parallel_loop(0, nv, unroll=4)
            def _p(j, li=li, shift=shift, want=want):
                key = load_keys(j)
                if li == 0:
                    b = lax.shift_right_arithmetic(key, 24) + 128
                    plsc.addupdate_scatter(hist_v, [lane_base + b], ones)
                else:
                    m = lax.shift_right_arithmetic(key, shift + 8) == want
                    b = lax.shift_right_arithmetic(key, shift) & 255
                    plsc.addupdate_scatter(hist_v, [lane_base + b], ones,
                                           mask=m)

            tn, above_n = scan_bins(rank)
            rank = rank - above_n
            want = tn - 128 if li == 0 else lax.shift_left(want, 8) | tn
        return want

    def process_row(r, c):
        pltpu.sync_copy(pre_hbm.at[base + r], row_v)

        @plsc.parallel_loop(0, ncv, unroll=2)
        def _ck(j):
            mx = jnp.full((16,), jnp.int32(-2147483648))
            for l in range(16):
                v = row_v[pl.ds(j * 16 + l * nvr, 16)]
                t = plsc.bitcast(v, jnp.int32)
                key = t ^ (lax.shift_right_arithmetic(t, 31)
                           & jnp.int32(0x7FFFFFFF))
                row_v[pl.ds(j * 16 + l * nvr, 16)] = plsc.bitcast(
                    key, jnp.float32)
                mx = jnp.maximum(mx, key)
            cm_v[pl.ds(j * 16, 16)] = mx

        def load_cm(j):
            return cm_v[pl.ds(j * 16, 16)]
        want_a = radix_select(ncv, load_cm, K_SPARSITY, 2)
        mhat = lax.shift_left(want_a, 16)

        @plsc.parallel_loop(0, ncv, unroll=8)
        def _pc(j):
            m = cm_v[pl.ds(j * 16, 16)] >= mhat
            cnt = jnp.full((16,), jnp.sum(m.astype(jnp.int32)), jnp.int32)
            plsc.store_scatter(cnt_v, [jnp.full((16,), j, jnp.int32)], cnt,
                               mask=(iota == 0))

        carry = jnp.int32(0)
        for v in range(ncv // 16):
            cv = cnt_v[pl.ds(v * 16, 16)]
            pc = plsc.cumsum(cv)
            base_v[pl.ds(v * 16, 16)] = pc - cv + carry
            carry = carry + jnp.max(pc)
        n_cand = carry

        @plsc.parallel_loop(0, ncv, unroll=4)
        def _cc(j):
            ck = cm_v[pl.ds(j * 16, 16)]
            m = ck >= mhat
            cs = plsc.cumsum(m.astype(jnp.int32))
            bj = plsc.load_gather(base_v, [jnp.full((16,), j, jnp.int32)])
            plsc.store_scatter(cid_v, [bj + cs - 1], j * 16 + iota, mask=m)

        @plsc.parallel_loop(0, n_cand, unroll=2)
        def _g(i):
            cid = plsc.load_gather(cid_v, [jnp.full((16,), i, jnp.int32)])
            keys = plsc.load_gather(row_v, [cid + iota * nvr])
            cand_v[pl.ds(i * 16, 16)] = keys

        def load_cand(j):
            return plsc.bitcast(cand_v[pl.ds(j * 16, 16)], jnp.int32)
        want = radix_select(n_cand, load_cand, K_SPARSITY, 4)

        kv = jnp.full((16,), want, dtype=jnp.int32)
        fv = plsc.bitcast(
            kv ^ (lax.shift_right_arithmetic(kv, 31) & jnp.int32(0x7FFFFFFF)),
            jnp.float32)
        plsc.store_scatter(th_v, [jnp.full((16,), r, jnp.int32)], fv,
                           mask=(iota == 0))
        return c

    lax.fori_loop(0, rows_per_w, process_row, 0)
    pltpu.sync_copy(th_v, th_hbm.at[pl.ds(base, rows_per_w)])


def _finish_body(pre_ref, th_ref, w_ref, pb_ref, mu_ref, std_ref,
                 lat_ref, rec_ref, *, n_lat_blocks):
    l = pl.program_id(1)
    pre = pre_ref[...]
    lat = jnp.where(pre >= th_ref[...], pre, 0.0)
    lat_ref[...] = lat
    part = jnp.dot(lat, w_ref[...], preferred_element_type=jnp.float32)

    @pl.when(l == 0)
    def _():
        rec_ref[...] = part

    @pl.when(l > 0)
    def _():
        rec_ref[...] += part

    @pl.when(l == n_lat_blocks - 1)
    def _():
        rec_ref[...] = (rec_ref[...] + pb_ref[...]) * std_ref[...] + mu_ref[...]


def kernel(pos, vel, acc, root_lin_vel, root_ang_vel, root_lin_acc,
           root_ang_acc, joint_centers, root_pos_history, root_euler_history,
           pre_bias, latent_bias, W_enc, W_dec):
    x = jnp.concatenate([
        pos, vel, acc, root_lin_vel, root_ang_vel, root_lin_acc, root_ang_acc,
        joint_centers, root_pos_history, root_euler_history,
    ], axis=-1)
    b = x.shape[0]
    pb = pre_bias.reshape(1, D_IN)
    lb = latent_bias.reshape(1, N_LATENTS)

    xc, mu, std = pl.pallas_call(
        _norm_body,
        grid=(b // BM0,),
        in_specs=[
            pl.BlockSpec((BM0, D_IN), lambda i: (i, 0)),
            pl.BlockSpec((1, D_IN), lambda i: (0, 0)),
        ],
        out_specs=[
            pl.BlockSpec((BM0, D_IN), lambda i: (i, 0)),
            pl.BlockSpec((BM0, 1), lambda i: (i, 0)),
            pl.BlockSpec((BM0, 1), lambda i: (i, 0)),
        ],
        out_shape=[
            jax.ShapeDtypeStruct((b, D_IN), jnp.float32),
            jax.ShapeDtypeStruct((b, 1), jnp.float32),
            jax.ShapeDtypeStruct((b, 1), jnp.float32),
        ],
    )(x, pb)

    pre_act = pl.pallas_call(
        _enc_body,
        grid=(N_LATENTS // LNE, b // BME),
        in_specs=[
            pl.BlockSpec((BME, D_IN), lambda l, i: (i, 0)),
            pl.BlockSpec((D_IN, LNE), lambda l, i: (0, l)),
            pl.BlockSpec((1, LNE), lambda l, i: (0, l)),
        ],
        out_specs=pl.BlockSpec((BME, LNE), lambda l, i: (i, l)),
        out_shape=jax.ShapeDtypeStruct((b, N_LATENTS), jnp.float32),
    )(xc, W_enc, lb)

    info = plsc.get_sparse_core_info()
    nw = info.num_cores * info.num_subcores
    rows_per_w = b // nw
    sel = pl.kernel(
        functools.partial(_sc_select_body, rows_per_w=rows_per_w,
                          num_cores=info.num_cores),
        out_type=jax.ShapeDtypeStruct((b,), jnp.float32),
        mesh=plsc.VectorSubcoreMesh(core_axis_name="c",
                                    subcore_axis_name="s"),
        compiler_params=pltpu.CompilerParams(needs_layout_passes=False),
        scratch_types=[
            pltpu.VMEM((N_LATENTS,), jnp.float32),
            pltpu.VMEM((N_LATENTS // 16,), jnp.int32),
            pltpu.VMEM((N_LATENTS // 256,), jnp.int32),
            pltpu.VMEM((N_LATENTS // 256,), jnp.int32),
            pltpu.VMEM((N_LATENTS // 16,), jnp.int32),
            pltpu.VMEM((N_LATENTS,), jnp.float32),
            pltpu.VMEM((4096,), jnp.int32),
            pltpu.VMEM((256,), jnp.int32),
            pltpu.VMEM((rows_per_w,), jnp.float32),
        ],
    )
    thresh = sel(pre_act).reshape(b, 1)

    n_lat_blocks = N_LATENTS // LN3
    latents, recons = pl.pallas_call(
        functools.partial(_finish_body, n_lat_blocks=n_lat_blocks),
        grid=(b // BM3, n_lat_blocks),
        in_specs=[
            pl.BlockSpec((BM3, LN3), lambda i, l: (i, l)),
            pl.BlockSpec((BM3, 1), lambda i, l: (i, 0)),
            pl.BlockSpec((LN3, D_IN), lambda i, l: (l, 0)),
            pl.BlockSpec((1, D_IN), lambda i, l: (0, 0)),
            pl.BlockSpec((BM3, 1), lambda i, l: (i, 0)),
            pl.BlockSpec((BM3, 1), lambda i, l: (i, 0)),
        ],
        out_specs=[
            pl.BlockSpec((BM3, LN3), lambda i, l: (i, l)),
            pl.BlockSpec((BM3, D_IN), lambda i, l: (i, 0)),
        ],
        out_shape=[
            jax.ShapeDtypeStruct((b, N_LATENTS), jnp.float32),
            jax.ShapeDtypeStruct((b, D_IN), jnp.float32),
        ],
    )(pre_act, thresh, W_dec, pb, mu, std)

    return pre_act, latents, recons


# speedup vs baseline: 1.4660x; 1.1570x over previous
import functools

import jax
import jax.numpy as jnp
from jax import lax
from jax.experimental import pallas as pl
from jax.experimental.pallas import tpu as pltpu
from jax.experimental.pallas import tpu_sc as plsc

B = 4096
D_IN = 210
N_LATENTS = 32768
K_SPARSITY = 100

BM0 = 256
BME, LNE = 256, 2048
BM3, LN3 = 256, 2048


def _norm_body(x_ref, pb_ref, xc_ref, mu_ref, std_ref):
    x = x_ref[...]
    mu = jnp.mean(x, axis=1, keepdims=True)
    std = jnp.sqrt(jnp.mean((x - mu) ** 2, axis=1, keepdims=True))
    xn = (x - mu) / (std + 1e-5)
    xc_ref[...] = xn - pb_ref[...]
    mu_ref[...] = mu
    std_ref[...] = std


def _enc_body(xc_ref, w_ref, b_ref, out_ref):
    out_ref[...] = (
        jnp.dot(xc_ref[...], w_ref[...], preferred_element_type=jnp.float32)
        + b_ref[...]
    )


def _sc_select_body(pre_hbm, th_hbm, row_v, cm_v, cnt_v, base_v, cid_v,
                    cand_v, hist_v, tot_v, th_v, *, rows_per_w, num_cores):
    wid = lax.axis_index("s") * num_cores + lax.axis_index("c")
    base = wid * rows_per_w
    iota = lax.iota(jnp.int32, 16)
    lane_base = iota * 256
    ones = jnp.ones((16,), jnp.int32)
    zeros16 = jnp.zeros((16,), jnp.int32)
    nvr = N_LATENTS // 16
    ncv = nvr // 16

    def scan_bins(rank):
        for g in range(16):
            acc = zeros16
            for l in range(16):
                acc = acc + hist_v[pl.ds(l * 256 + g * 16, 16)]
                hist_v[pl.ds(l * 256 + g * 16, 16)] = zeros16
            tot_v[pl.ds(g * 16, 16)] = acc
        trs = [plsc.load_gather(tot_v, [iota * 16 + t]) for t in range(16)]
        vs = [zeros16] * 16
        run = zeros16
        for t in range(15, -1, -1):
            run = run + trs[t]
            vs[t] = run
        pc = plsc.cumsum(vs[0])
        above_vec = jnp.max(pc) - pc
        cnt_vec = zeros16
        sel_vec = zeros16
        for t in range(16):
            cond = (above_vec + vs[t]) >= rank
            cnt_vec = cnt_vec + cond.astype(jnp.int32)
            sel_vec = sel_vec + jnp.where(cond, 0, trs[t])
        T = jnp.sum(cnt_vec) - 1
        above_sel = jnp.sum(sel_vec)
        return T, above_sel

    def radix_select(nv, load_keys, k_rank, levels):
        rank = k_rank
        want = jnp.int32(0)
        for li, shift in enumerate((24, 16, 8, 0)[:levels]):
            @plsc.parallel_loop(0, nv, unroll=4)
            def _p(j, li=li, shift=shift, want=want):
                key = load_keys(j)
                if li == 0:
                    b = lax.shift_right_arithmetic(key, 24) + 128
                    plsc.addupdate_scatter(hist_v, [lane_base + b], ones)
                else:
                    m = lax.shift_right_arithmetic(key, shift + 8) == want
                    b = lax.shift_right_arithmetic(key, shift) & 255
                    plsc.addupdate_scatter(hist_v, [lane_base + b], ones,
                                           mask=m)

            tn, above_n = scan_bins(rank)
            rank = rank - above_n
            want = tn - 128 if li == 0 else lax.shift_left(want, 8) | tn
        return want

    def process_row(r, c):
        pltpu.sync_copy(pre_hbm.at[base + r], row_v)

        @plsc.parallel_loop(0, ncv, unroll=2)
        def _ck(j):
            hist_v[pl.ds(j * 32, 16)] = zeros16
            hist_v[pl.ds(j * 32 + 16, 16)] = zeros16
            mx = jnp.full((16,), jnp.int32(-2147483648))
            for l in range(16):
                v = row_v[pl.ds(j * 16 + l * nvr, 16)]
                t = plsc.bitcast(v, jnp.int32)
                key = t ^ (lax.shift_right_arithmetic(t, 31)
                           & jnp.int32(0x7FFFFFFF))
                row_v[pl.ds(j * 16 + l * nvr, 16)] = plsc.bitcast(
                    key, jnp.float32)
                mx = jnp.maximum(mx, key)
            cm_v[pl.ds(j * 16, 16)] = mx

        def load_cm(j):
            return cm_v[pl.ds(j * 16, 16)]
        want_a = radix_select(ncv, load_cm, K_SPARSITY, 2)
        mhat = lax.shift_left(want_a, 16)

        @plsc.parallel_loop(0, ncv, unroll=8)
        def _pc(j):
            m = cm_v[pl.ds(j * 16, 16)] >= mhat
            cnt = jnp.full((16,), jnp.sum(m.astype(jnp.int32)), jnp.int32)
            plsc.store_scatter(cnt_v, [jnp.full((16,), j, jnp.int32)], cnt,
                               mask=(iota == 0))

        carry = jnp.int32(0)
        for v in range(ncv // 16):
            cv = cnt_v[pl.ds(v * 16, 16)]
            pc = plsc.cumsum(cv)
            base_v[pl.ds(v * 16, 16)] = pc - cv + carry
            carry = carry + jnp.max(pc)
        n_cand = carry

        @plsc.parallel_loop(0, ncv, unroll=4)
        def _cc(j):
            ck = cm_v[pl.ds(j * 16, 16)]
            m = ck >= mhat
            cs = plsc.cumsum(m.astype(jnp.int32))
            bj = plsc.load_gather(base_v, [jnp.full((16,), j, jnp.int32)])
            plsc.store_scatter(cid_v, [bj + cs - 1], j * 16 + iota, mask=m)

        @plsc.parallel_loop(0, n_cand, unroll=2)
        def _g(i):
            cid = plsc.load_gather(cid_v, [jnp.full((16,), i, jnp.int32)])
            keys = plsc.load_gather(row_v, [cid + iota * nvr])
            cand_v[pl.ds(i * 16, 16)] = keys

        def load_cand(j):
            return plsc.bitcast(cand_v[pl.ds(j * 16, 16)], jnp.int32)
        want = radix_select(n_cand, load_cand, K_SPARSITY, 4)

        kv = jnp.full((16,), want, dtype=jnp.int32)
        fv = plsc.bitcast(
            kv ^ (lax.shift_right_arithmetic(kv, 31) & jnp.int32(0x7FFFFFFF)),
            jnp.float32)
        plsc.store_scatter(th_v, [jnp.full((16,), r, jnp.int32)], fv,
                           mask=(iota == 0))
        return c

    lax.fori_loop(0, rows_per_w, process_row, 0)
    pltpu.sync_copy(th_v, th_hbm.at[pl.ds(base, rows_per_w)])


def _finish_body(pre_ref, th_ref, w_ref, pb_ref, mu_ref, std_ref,
                 lat_ref, rec_ref, *, n_lat_blocks):
    l = pl.program_id(1)
    pre = pre_ref[...]
    lat = jnp.where(pre >= th_ref[...], pre, 0.0)
    lat_ref[...] = lat
    part = jnp.dot(lat, w_ref[...], preferred_element_type=jnp.float32)

    @pl.when(l == 0)
    def _():
        rec_ref[...] = part

    @pl.when(l > 0)
    def _():
        rec_ref[...] += part

    @pl.when(l == n_lat_blocks - 1)
    def _():
        rec_ref[...] = (rec_ref[...] + pb_ref[...]) * std_ref[...] + mu_ref[...]


def kernel(pos, vel, acc, root_lin_vel, root_ang_vel, root_lin_acc,
           root_ang_acc, joint_centers, root_pos_history, root_euler_history,
           pre_bias, latent_bias, W_enc, W_dec):
    x = jnp.concatenate([
        pos, vel, acc, root_lin_vel, root_ang_vel, root_lin_acc, root_ang_acc,
        joint_centers, root_pos_history, root_euler_history,
    ], axis=-1)
    b = x.shape[0]
    pb = pre_bias.reshape(1, D_IN)
    lb = latent_bias.reshape(1, N_LATENTS)

    xc, mu, std = pl.pallas_call(
        _norm_body,
        grid=(b // BM0,),
        in_specs=[
            pl.BlockSpec((BM0, D_IN), lambda i: (i, 0)),
            pl.BlockSpec((1, D_IN), lambda i: (0, 0)),
        ],
        out_specs=[
            pl.BlockSpec((BM0, D_IN), lambda i: (i, 0)),
            pl.BlockSpec((BM0, 1), lambda i: (i, 0)),
            pl.BlockSpec((BM0, 1), lambda i: (i, 0)),
        ],
        out_shape=[
            jax.ShapeDtypeStruct((b, D_IN), jnp.float32),
            jax.ShapeDtypeStruct((b, 1), jnp.float32),
            jax.ShapeDtypeStruct((b, 1), jnp.float32),
        ],
    )(x, pb)

    pre_act = pl.pallas_call(
        _enc_body,
        grid=(N_LATENTS // LNE, b // BME),
        in_specs=[
            pl.BlockSpec((BME, D_IN), lambda l, i: (i, 0)),
            pl.BlockSpec((D_IN, LNE), lambda l, i: (0, l)),
            pl.BlockSpec((1, LNE), lambda l, i: (0, l)),
        ],
        out_specs=pl.BlockSpec((BME, LNE), lambda l, i: (i, l)),
        out_shape=jax.ShapeDtypeStruct((b, N_LATENTS), jnp.float32),
    )(xc, W_enc, lb)

    info = plsc.get_sparse_core_info()
    nw = info.num_cores * info.num_subcores
    rows_per_w = b // nw
    sel = pl.kernel(
        functools.partial(_sc_select_body, rows_per_w=rows_per_w,
                          num_cores=info.num_cores),
        out_type=jax.ShapeDtypeStruct((b,), jnp.float32),
        mesh=plsc.VectorSubcoreMesh(core_axis_name="c",
                                    subcore_axis_name="s"),
        compiler_params=pltpu.CompilerParams(needs_layout_passes=False),
        scratch_types=[
            pltpu.VMEM((N_LATENTS,), jnp.float32),
            pltpu.VMEM((N_LATENTS // 16,), jnp.int32),
            pltpu.VMEM((N_LATENTS // 256,), jnp.int32),
            pltpu.VMEM((N_LATENTS // 256,), jnp.int32),
            pltpu.VMEM((N_LATENTS // 16,), jnp.int32),
            pltpu.VMEM((N_LATENTS,), jnp.float32),
            pltpu.VMEM((4096,), jnp.int32),
            pltpu.VMEM((256,), jnp.int32),
            pltpu.VMEM((rows_per_w,), jnp.float32),
        ],
    )
    thresh = sel(pre_act).reshape(b, 1)

    n_lat_blocks = N_LATENTS // LN3
    latents, recons = pl.pallas_call(
        functools.partial(_finish_body, n_lat_blocks=n_lat_blocks),
        grid=(b // BM3, n_lat_blocks),
        in_specs=[
            pl.BlockSpec((BM3, LN3), lambda i, l: (i, l)),
            pl.BlockSpec((BM3, 1), lambda i, l: (i, 0)),
            pl.BlockSpec((LN3, D_IN), lambda i, l: (l, 0)),
            pl.BlockSpec((1, D_IN), lambda i, l: (0, 0)),
            pl.BlockSpec((BM3, 1), lambda i, l: (i, 0)),
            pl.BlockSpec((BM3, 1), lambda i, l: (i, 0)),
        ],
        out_specs=[
            pl.BlockSpec((BM3, LN3), lambda i, l: (i, l)),
            pl.BlockSpec((BM3, D_IN), lambda i, l: (i, 0)),
        ],
        out_shape=[
            jax.ShapeDtypeStruct((b, N_LATENTS), jnp.float32),
            jax.ShapeDtypeStruct((b, D_IN), jnp.float32),
        ],
    )(pre_act, thresh, W_dec, pb, mu, std)

    return pre_act, latents, recons
